# baseline (speedup 1.0000x reference)
import functools

import jax
import jax.numpy as jnp
from jax import lax
from jax.experimental import pallas as pl
from jax.experimental.pallas import tpu as pltpu
from jax.experimental.pallas import tpu_sc as plsc

BATCH = 16384
DIM = 4
LANES = 16
TILE_ROWS = 128
NUM_WORKERS = 32
B_PER_W = BATCH // NUM_WORKERS
N_VEC = B_PER_W // LANES


def _rsqrt16(x):
    i = lax.bitcast_convert_type(x, jnp.int32)
    y = lax.bitcast_convert_type(jnp.int32(0x5F3759DF) - (i >> 1), jnp.float32)
    for _ in range(3):
        y = y * (jnp.float32(1.5) - jnp.float32(0.5) * x * y * y)
    return y


@functools.cache
def _make_sc_kernel(n_main, n_tail):
    mesh = plsc.VectorSubcoreMesh(core_axis_name="c", subcore_axis_name="s")

    @functools.partial(
        pl.kernel,
        mesh=mesh,
        out_type=jax.ShapeDtypeStruct((BATCH,), jnp.float32),
        compiler_params=pltpu.CompilerParams(
            needs_layout_passes=False, use_tc_tiling_on_sc=False
        ),
        scratch_types=[
            pltpu.VMEM((B_PER_W,), jnp.int32),
            pltpu.VMEM((B_PER_W,), jnp.int32),
            pltpu.VMEM((DIM * B_PER_W,), jnp.int32),
            pltpu.VMEM((DIM * B_PER_W,), jnp.int32),
            pltpu.VMEM((DIM * B_PER_W,), jnp.float32),
            pltpu.VMEM((DIM * B_PER_W,), jnp.float32),
            pltpu.VMEM((DIM * n_tail,), jnp.float32),
            pltpu.VMEM((DIM * n_tail,), jnp.float32),
            pltpu.VMEM((B_PER_W,), jnp.float32),
            pltpu.SemaphoreType.DMA,
        ],
    )
    def sc_kernel(uin_hbm, iin_hbm, umain_hbm, utail_hbm, imain_hbm,
                  itail_hbm, out_hbm, uidx, iidx, uoff, ioff, ucomp, icomp,
                  utail, itail, outv, sem):
        wid = lax.axis_index("s") * 2 + lax.axis_index("c")
        base = wid * B_PER_W

        pltpu.sync_copy(uin_hbm.at[pl.ds(base, B_PER_W)], uidx)
        pltpu.sync_copy(iin_hbm.at[pl.ds(base, B_PER_W)], iidx)
        pltpu.sync_copy(utail_hbm, utail)
        pltpu.sync_copy(itail_hbm, itail)

        for c in range(N_VEC):
            sl = pl.ds(c * LANES, LANES)
            for idx, off in ((uidx, uoff), (iidx, ioff)):
                i = jnp.minimum(idx[sl], n_main - 1)
                base_off = ((i >> 7) << 9) + (i & 127)
                for d in range(DIM):
                    off[pl.ds(d * B_PER_W + c * LANES, LANES)] = (
                        base_off + (d << 7))

        copies = []
        for d in range(DIM):
            sl = pl.ds(d * B_PER_W, B_PER_W)
            copies.append(pltpu.make_async_copy(
                umain_hbm.at[uoff.at[sl]], ucomp.at[sl], sem))
            copies.append(pltpu.make_async_copy(
                imain_hbm.at[ioff.at[sl]], icomp.at[sl], sem))
        for cp in copies:
            cp.start()
        for cp in copies:
            cp.wait()

        for c in range(N_VEC):
            sl = pl.ds(c * LANES, LANES)
            ui = uidx[sl]
            ii = iidx[sl]
            u_tail_sel = ui >= n_main
            i_tail_sel = ii >= n_main
            ut4 = (jnp.clip(ui - n_main, 0, n_tail - 1) << 2)
            it4 = (jnp.clip(ii - n_main, 0, n_tail - 1) << 2)
            ud, vd = [], []
            for d in range(DIM):
                um = ucomp[pl.ds(d * B_PER_W + c * LANES, LANES)]
                im = icomp[pl.ds(d * B_PER_W + c * LANES, LANES)]
                ut = plsc.load_gather(utail, [ut4 + d])
                it = plsc.load_gather(itail, [it4 + d])
                ud.append(jnp.where(u_tail_sel, ut, um))
                vd.append(jnp.where(i_tail_sel, it, im))
            dot = ud[0] * vd[0]
            nu = ud[0] * ud[0]
            nv = vd[0] * vd[0]
            for d in range(1, DIM):
                dot = dot + ud[d] * vd[d]
                nu = nu + ud[d] * ud[d]
                nv = nv + vd[d] * vd[d]
            outv[pl.ds(c * LANES, LANES)] = dot * _rsqrt16(nu) * _rsqrt16(nv)

        pltpu.sync_copy(outv, out_hbm.at[pl.ds(base, B_PER_W)])

    return sc_kernel


def _views(table):
    n = table.shape[0]
    n_main = (n // TILE_ROWS) * TILE_ROWS
    main = (table.T[:, :n_main]
            .reshape(DIM, n_main // TILE_ROWS, TILE_ROWS)
            .transpose(1, 0, 2)
            .reshape(-1))
    tail = table[n_main:].reshape(-1)
    return main, tail, n_main, n - n_main


def kernel(user_input, item_input, user_table, item_table):
    umain, utail, n_main, n_tail = _views(user_table)
    imain, itail, _, _ = _views(item_table)
    sc = _make_sc_kernel(n_main, n_tail)
    return sc(user_input, item_input, umain, utail, imain, itail)


# speedup vs baseline: 1.5580x; 1.5580x over previous
import functools

import jax
import jax.numpy as jnp
from jax import lax
from jax.experimental import pallas as pl
from jax.experimental.pallas import tpu as pltpu
from jax.experimental.pallas import tpu_sc as plsc

BATCH = 16384
DIM = 4
LANES = 16
NUM_WORKERS = 32
B_PER_W = BATCH // NUM_WORKERS
N_VEC = B_PER_W // LANES


def _rsqrt16(x):
    i = lax.bitcast_convert_type(x, jnp.int32)
    y = lax.bitcast_convert_type(jnp.int32(0x5F3759DF) - (i >> 1), jnp.float32)
    for _ in range(3):
        y = y * (jnp.float32(1.5) - jnp.float32(0.5) * x * y * y)
    return y


mesh = plsc.VectorSubcoreMesh(core_axis_name="c", subcore_axis_name="s")


@functools.partial(
    pl.kernel,
    mesh=mesh,
    out_type=jax.ShapeDtypeStruct((BATCH,), jnp.float32),
    compiler_params=pltpu.CompilerParams(
        needs_layout_passes=False, use_tc_tiling_on_sc=False
    ),
    scratch_types=[
        pltpu.VMEM((B_PER_W,), jnp.int32),
        pltpu.VMEM((B_PER_W,), jnp.int32),
        pltpu.VMEM((DIM * B_PER_W,), jnp.float32),
        pltpu.VMEM((DIM * B_PER_W,), jnp.float32),
        pltpu.VMEM((B_PER_W,), jnp.float32),
        pltpu.SemaphoreType.DMA,
    ],
)
def _sc_kernel(uin_hbm, iin_hbm, utab_hbm, itab_hbm, out_hbm,
               uidx, iidx, ucomp, icomp, outv, sem):
    wid = lax.axis_index("s") * 2 + lax.axis_index("c")
    base = wid * B_PER_W
    nrows = utab_hbm.shape[0] // DIM

    pltpu.sync_copy(uin_hbm.at[pl.ds(base, B_PER_W)], uidx)
    pltpu.sync_copy(iin_hbm.at[pl.ds(base, B_PER_W)], iidx)

    copies = []
    for d in range(DIM):
        sl = pl.ds(d * B_PER_W, B_PER_W)
        usrc = utab_hbm.at[pl.ds(d * nrows, nrows)]
        isrc = itab_hbm.at[pl.ds(d * nrows, nrows)]
        copies.append(pltpu.make_async_copy(usrc.at[uidx], ucomp.at[sl], sem))
        copies.append(pltpu.make_async_copy(isrc.at[iidx], icomp.at[sl], sem))
    for cp in copies:
        cp.start()
    for cp in copies:
        cp.wait()

    for c in range(N_VEC):
        ud = [ucomp[pl.ds(d * B_PER_W + c * LANES, LANES)] for d in range(DIM)]
        vd = [icomp[pl.ds(d * B_PER_W + c * LANES, LANES)] for d in range(DIM)]
        dot = ud[0] * vd[0]
        nu = ud[0] * ud[0]
        nv = vd[0] * vd[0]
        for d in range(1, DIM):
            dot = dot + ud[d] * vd[d]
            nu = nu + ud[d] * ud[d]
            nv = nv + vd[d] * vd[d]
        outv[pl.ds(c * LANES, LANES)] = dot * _rsqrt16(nu) * _rsqrt16(nv)

    pltpu.sync_copy(outv, out_hbm.at[pl.ds(base, B_PER_W)])


def kernel(user_input, item_input, user_table, item_table):
    return _sc_kernel(user_input, item_input,
                      user_table.T.reshape(-1), item_table.T.reshape(-1))
